# baseline (speedup 1.0000x reference)
import functools

import jax
import jax.numpy as jnp
from jax import lax
from jax.experimental import pallas as pl
from jax.experimental.pallas import tpu as pltpu
from jax.experimental.pallas import tpu_sc as plsc

_PIECE = 16384
_NB = 5
_LOOK = 3
_CB = 16384


class _Ring:
    def __init__(self, buf, sins, souts, src_slice, dst_slice, n):
        self.buf, self.sins, self.souts = buf, sins, souts
        self.src, self.dst, self.n = src_slice, dst_slice, n
        self.gh = [None] * _NB
        self.sh = [None] * _NB

    def _gather(self, k):
        b = k % _NB
        if self.sh[b] is not None:
            self.sh[b].wait()
            self.sh[b] = None
        self.gh[b] = pltpu.async_copy(
            self.src(k), self.buf.at[pl.ds(b * _PIECE, _PIECE)], self.sins[b]
        )

    def prime(self):
        for k in range(min(_LOOK, self.n)):
            self._gather(k)

    def step(self, k):
        b = k % _NB
        self.gh[b].wait()
        self.sh[b] = pltpu.async_copy(
            self.buf.at[pl.ds(b * _PIECE, _PIECE)], self.dst(k), self.souts[b]
        )
        if k + _LOOK < self.n:
            self._gather(k + _LOOK)

    def drain(self):
        for b in range(_NB):
            if self.sh[b] is not None:
                self.sh[b].wait()


_ONES = 16384


def _sc_permute_indices_and_lengths(values, W, C):
    info = plsc.get_sparse_core_info()
    nc, ns = info.num_cores, info.num_subcores
    nw = nc * ns
    per_w = (W * W) // nw
    npieces = C // _PIECE
    n = per_w * npieces
    N = W * C
    lseg = (W * N) // nw
    per_row = N // lseg
    nlscat = lseg // _ONES

    mesh = plsc.VectorSubcoreMesh(core_axis_name="c", subcore_axis_name="s")

    @functools.partial(
        pl.kernel,
        mesh=mesh,
        out_type=[
            jax.ShapeDtypeStruct((W, N), jnp.int32),
            jax.ShapeDtypeStruct((W, N), jnp.int32),
        ],
        scratch_types=[
            pltpu.VMEM((_NB * _PIECE,), jnp.int32),
            pltpu.VMEM((_ONES,), jnp.int32),
        ]
        + [pltpu.SemaphoreType.DMA] * (2 * _NB + 1),
    )
    def k(vals_hbm, out_hbm, len_hbm, vbuf, ones_buf, *sems):
        sin, sout, slen = sems[:_NB], sems[_NB : 2 * _NB], sems[2 * _NB]
        wid = lax.axis_index("s") * nc + lax.axis_index("c")
        coords = []
        for t in range(per_w):
            p = wid * per_w + t
            coords.append((p // W, p % W))

        def src(k_):
            s, w = coords[k_ // npieces]
            return vals_hbm.at[s, w, pl.ds((k_ % npieces) * _PIECE, _PIECE)]

        def dst(k_):
            s, w = coords[k_ // npieces]
            return out_hbm.at[w, pl.ds(s * C + (k_ % npieces) * _PIECE, _PIECE)]

        def fill(i, _):
            ones_buf[pl.ds(i * 16, 16)] = jnp.ones((16,), jnp.int32)
            return 0

        lax.fori_loop(0, _ONES // 16, fill, 0)
        lrow = wid // per_row
        lcol = (wid % per_row) * lseg
        lhs = []
        for j in range(nlscat):
            lhs.append(
                pltpu.async_copy(
                    ones_buf, len_hbm.at[lrow, pl.ds(lcol + j * _ONES, _ONES)], slen
                )
            )

        ring = _Ring(vbuf, sin, sout, src, dst, n)
        ring.prime()
        for kk in range(n):
            ring.step(kk)
        ring.drain()
        for h in lhs:
            h.wait()

    return k(values)


def _tc_weights_permute_and_offsets(weights, W, C, N):
    cb = C
    ocb = ((N + W) // W + 127) // 128 * 128

    def body(w_ref, out_w_ref, off_ref):
        i = pl.program_id(0)
        out_w_ref[...] = w_ref[0]
        off_ref[...] = i * ocb + lax.broadcasted_iota(jnp.int32, (W, ocb), 1)

    return pl.pallas_call(
        body,
        grid=(W,),
        in_specs=[pl.BlockSpec((1, W, cb), lambda i: (i, 0, 0))],
        out_specs=[
            pl.BlockSpec((W, cb), lambda i: (0, i)),
            pl.BlockSpec((W, ocb), lambda i: (0, i)),
        ],
        out_shape=[
            jax.ShapeDtypeStruct((W, W * C), jnp.float32),
            jax.ShapeDtypeStruct((W, N + 1), jnp.int32),
        ],
    )(weights)


def kernel(values, weights):
    W, _, C = values.shape
    N = W * C
    out_indices, kjt_lengths = _sc_permute_indices_and_lengths(values, W, C)
    out_weights, kjt_offsets = _tc_weights_permute_and_offsets(weights, W, C, N)
    return out_indices, out_weights, kjt_lengths, kjt_offsets


# speedup vs baseline: 1.0045x; 1.0045x over previous
import functools

import jax
import jax.numpy as jnp
from jax import lax
from jax.experimental import pallas as pl
from jax.experimental.pallas import tpu as pltpu
from jax.experimental.pallas import tpu_sc as plsc

_PIECE = 16384
_NB = 5
_LOOK = 3
_CB = 16384


class _Ring:
    def __init__(self, buf, sins, souts, src_slice, dst_slice, n):
        self.buf, self.sins, self.souts = buf, sins, souts
        self.src, self.dst, self.n = src_slice, dst_slice, n
        self.gh = [None] * _NB
        self.sh = [None] * _NB

    def _gather(self, k):
        b = k % _NB
        if self.sh[b] is not None:
            self.sh[b].wait()
            self.sh[b] = None
        self.gh[b] = pltpu.async_copy(
            self.src(k), self.buf.at[pl.ds(b * _PIECE, _PIECE)], self.sins[b]
        )

    def prime(self):
        for k in range(min(_LOOK, self.n)):
            self._gather(k)

    def step(self, k):
        b = k % _NB
        self.gh[b].wait()
        self.sh[b] = pltpu.async_copy(
            self.buf.at[pl.ds(b * _PIECE, _PIECE)], self.dst(k), self.souts[b]
        )
        if k + _LOOK < self.n:
            self._gather(k + _LOOK)

    def drain(self):
        for b in range(_NB):
            if self.sh[b] is not None:
                self.sh[b].wait()


_ONES = 16384


def _sc_permute_indices_and_lengths(values, W, C):
    info = plsc.get_sparse_core_info()
    nc, ns = info.num_cores, info.num_subcores
    nw = nc * ns
    per_w = (W * W) // nw
    npieces = C // _PIECE
    n = per_w * npieces
    N = W * C
    lseg = (W * N) // nw
    per_row = N // lseg
    nlscat = lseg // _ONES

    mesh = plsc.VectorSubcoreMesh(core_axis_name="c", subcore_axis_name="s")

    @functools.partial(
        pl.kernel,
        mesh=mesh,
        out_type=[
            jax.ShapeDtypeStruct((W, N), jnp.int32),
            jax.ShapeDtypeStruct((W, N), jnp.int32),
        ],
        scratch_types=[
            pltpu.VMEM((_NB * _PIECE,), jnp.int32),
            pltpu.VMEM((_ONES,), jnp.int32),
        ]
        + [pltpu.SemaphoreType.DMA] * (2 * _NB + 1),
    )
    def k(vals_hbm, out_hbm, len_hbm, vbuf, ones_buf, *sems):
        sin, sout, slen = sems[:_NB], sems[_NB : 2 * _NB], sems[2 * _NB]
        wid = lax.axis_index("s") * nc + lax.axis_index("c")
        coords = []
        for t in range(per_w):
            p = wid * per_w + t
            coords.append((p // W, p % W))

        def src(k_):
            s, w = coords[k_ // npieces]
            return vals_hbm.at[s, w, pl.ds((k_ % npieces) * _PIECE, _PIECE)]

        def dst(k_):
            s, w = coords[k_ // npieces]
            return out_hbm.at[w, pl.ds(s * C + (k_ % npieces) * _PIECE, _PIECE)]

        def fill(i, _):
            ones_buf[pl.ds(i * 16, 16)] = jnp.ones((16,), jnp.int32)
            return 0

        lax.fori_loop(0, _ONES // 16, fill, 0)
        lrow = wid // per_row
        lcol = (wid % per_row) * lseg
        lhs = []
        for j in range(nlscat):
            lhs.append(
                pltpu.async_copy(
                    ones_buf, len_hbm.at[lrow, pl.ds(lcol + j * _ONES, _ONES)], slen
                )
            )

        ring = _Ring(vbuf, sin, sout, src, dst, n)
        ring.prime()
        for kk in range(n):
            ring.step(kk)
        ring.drain()
        for h in lhs:
            h.wait()

    return k(values)


def _tc_weights_permute_and_offsets(weights, W, C, N):
    cb = C
    ocb = ((N + W) // W + 127) // 128 * 128

    def body(w_ref, out_w_ref, off_ref):
        i = pl.program_id(0)
        out_w_ref[...] = w_ref[0]
        off_ref[...] = i * ocb + lax.broadcasted_iota(jnp.int32, (W, ocb), 1)

    return pl.pallas_call(
        body,
        grid=(W,),
        in_specs=[pl.BlockSpec((1, W, cb), lambda i: (i, 0, 0))],
        out_specs=[
            pl.BlockSpec((W, cb), lambda i: (0, i)),
            pl.BlockSpec((W, ocb), lambda i: (0, i)),
        ],
        out_shape=[
            jax.ShapeDtypeStruct((W, W * C), jnp.float32),
            jax.ShapeDtypeStruct((W, N + 1), jnp.int32),
        ],
    )(weights)


def kernel(values, weights):
    W, _, C = values.shape
    N = W * C
    out_weights, kjt_offsets = _tc_weights_permute_and_offsets(weights, W, C, N)
    out_indices, kjt_lengths = _sc_permute_indices_and_lengths(values, W, C)
    return out_indices, out_weights, kjt_lengths, kjt_offsets
